# Initial kernel scaffold; baseline (speedup 1.0000x reference)
import jax
import jax.numpy as jnp
from jax.experimental import pallas as pl


def kernel(x, table, W1, b1, W2, b2, W3, b3, A1, a1, A2, a2, C1, c1, C2, c2):
    raise NotImplementedError("write your pallas kernel here")


# speedup vs baseline: 20.1967x; 20.1967x over previous
import functools

import jax
import jax.numpy as jnp
from jax import lax
from jax.experimental import pallas as pl
from jax.experimental.pallas import tpu as pltpu
from jax.experimental.pallas import tpu_sc as plsc

B, S, V, D = 4096, 200, 1000, 128
K = 20
P = 1024
R = 256
NC, NS = 2, 16
NW = NC * NS
CHUNK = (B * S) // NW
GL = 16


def _precompute_body(t_ref, a1m_ref, a1b_ref, a2m_ref, a2b_ref, att_ref):
    t = t_ref[...]
    a = jnp.dot(t, a1m_ref[...], preferred_element_type=jnp.float32)
    a = jnp.maximum(a + a1b_ref[...], 0.0)
    a = jnp.dot(a, a2m_ref[...], preferred_element_type=jnp.float32)
    att_ref[...] = a + a2b_ref[...]


def _precompute(table_p, A1, a1r, A2, a2r):
    return pl.pallas_call(
        _precompute_body,
        out_shape=jax.ShapeDtypeStruct((P, D), jnp.float32),
    )(table_p, A1, a1r, A2, a2r)


def _gather_body(sv_hbm, x_hbm, out_hbm, sv_v, idx_v, val_v):
    wid = lax.axis_index("s") * NC + lax.axis_index("c")
    base = wid * CHUNK
    pltpu.sync_copy(sv_hbm, sv_v)
    pltpu.sync_copy(x_hbm.at[pl.ds(base, CHUNK)], idx_v)

    def body(i, carry):
        o = i * GL
        idx = idx_v[pl.ds(o, GL)]
        val_v[pl.ds(o, GL)] = plsc.load_gather(sv_v, [idx])
        return carry

    lax.fori_loop(0, CHUNK // GL, body, 0)
    pltpu.sync_copy(val_v, out_hbm.at[pl.ds(base, CHUNK)])


def _score_gather(sv_flat, x_flat):
    mesh = plsc.VectorSubcoreMesh(core_axis_name="c", subcore_axis_name="s")
    fn = pl.kernel(
        _gather_body,
        out_type=jax.ShapeDtypeStruct((B * S,), jnp.float32),
        mesh=mesh,
        scratch_types=[
            pltpu.VMEM((P,), jnp.float32),
            pltpu.VMEM((CHUNK,), jnp.int32),
            pltpu.VMEM((CHUNK,), jnp.float32),
        ],
        compiler_params=pltpu.CompilerParams(needs_layout_passes=False),
    )
    return fn(sv_flat, x_flat)


def _topk_body(sc_ref, x_ref, att_ref, c1m_ref, c1b_ref, c2r_ref, c2b_ref,
               idx_ref, pred_ref):
    sc = sc_ref[...]
    xb = x_ref[...]
    pos = lax.broadcasted_iota(jnp.int32, (R, S), 1)
    viota = lax.broadcasted_iota(jnp.int32, (R, P), 1)
    cols = lax.broadcasted_iota(jnp.int32, (R, K), 1)
    m_cnt = jnp.zeros((R, P), jnp.float32)
    topidx = jnp.zeros((R, K), jnp.int32)
    for j in range(K):
        m = jnp.max(sc, axis=1, keepdims=True)
        idxj = jnp.min(jnp.where(sc == m, pos, S), axis=1, keepdims=True)
        onehot = pos == idxj
        selid = jnp.sum(jnp.where(onehot, xb, 0), axis=1, keepdims=True)
        m_cnt = m_cnt + jnp.where(viota == selid, 1.0, 0.0)
        topidx = jnp.where(cols == j, idxj, topidx)
        sc = jnp.where(onehot, -1e30, sc)
    pooled = jnp.dot(m_cnt, att_ref[...],
                     preferred_element_type=jnp.float32) * (1.0 / K)
    h = jnp.dot(pooled, c1m_ref[...], preferred_element_type=jnp.float32)
    h = jnp.maximum(h + c1b_ref[...], 0.0)
    logit = jnp.sum(h * c2r_ref[...], axis=1, keepdims=True) + c2b_ref[...]
    idx_ref[...] = topidx
    pred_ref[...] = jax.nn.sigmoid(logit)


def _topk(final_scores, x, att, C1, c1r, c2row, c2b):
    grid = B // R
    return pl.pallas_call(
        _topk_body,
        grid=(grid,),
        in_specs=[
            pl.BlockSpec((R, S), lambda i: (i, 0)),
            pl.BlockSpec((R, S), lambda i: (i, 0)),
            pl.BlockSpec((P, D), lambda i: (0, 0)),
            pl.BlockSpec((D, D // 2), lambda i: (0, 0)),
            pl.BlockSpec((1, D // 2), lambda i: (0, 0)),
            pl.BlockSpec((1, D // 2), lambda i: (0, 0)),
            pl.BlockSpec((1, 1), lambda i: (0, 0)),
        ],
        out_specs=(
            pl.BlockSpec((R, K), lambda i: (i, 0)),
            pl.BlockSpec((R, 1), lambda i: (i, 0)),
        ),
        out_shape=(
            jax.ShapeDtypeStruct((B, K), jnp.int32),
            jax.ShapeDtypeStruct((B, 1), jnp.float32),
        ),
    )(final_scores, x, att, C1, c1r, c2row, c2b)


def kernel(x, table, W1, b1, W2, b2, W3, b3, A1, a1, A2, a2, C1, c1, C2, c2):
    table_p = jnp.pad(table, ((0, P - V), (0, 0)))
    att = _precompute(table_p, A1, a1.reshape(1, -1), A2, a2.reshape(1, -1))
    h = jax.nn.relu(table @ W1 + b1)
    h = jax.nn.relu(h @ W2 + b2)
    sv = jax.nn.sigmoid((h @ W3 + b3).reshape(V))
    sv_flat = jnp.pad(sv, (0, P - V))
    scores_flat = _score_gather(sv_flat, x.reshape(B * S))
    final_scores = scores_flat.reshape(B, S)
    topidx, pred2d = _topk(final_scores, x, att, C1, c1.reshape(1, -1),
                           C2.reshape(1, -1), c2.reshape(1, 1))
    return (pred2d.reshape(B), topidx, final_scores)


# speedup vs baseline: 22.2568x; 1.1020x over previous
import functools

import jax
import jax.numpy as jnp
from jax import lax
from jax.experimental import pallas as pl
from jax.experimental.pallas import tpu as pltpu
from jax.experimental.pallas import tpu_sc as plsc

B, S, V, D = 4096, 200, 1000, 128
K = 20
P = 1024
R = 256
NC, NS = 2, 16
NW = NC * NS
CHUNK = (B * S) // NW
GL = 16
IMAX = 0x7FFFFFFF


def _gather_body(sv_hbm, rid_hbm, x_hbm, sc_out, rk_out,
                 sv_v, rid_v, idx_v, val_v, rk_v):
    wid = lax.axis_index("s") * NC + lax.axis_index("c")
    base = wid * CHUNK
    pltpu.sync_copy(sv_hbm, sv_v)
    pltpu.sync_copy(rid_hbm, rid_v)
    pltpu.sync_copy(x_hbm.at[pl.ds(base, CHUNK)], idx_v)

    def body(i, carry):
        o = i * GL
        idx = idx_v[pl.ds(o, GL)]
        val_v[pl.ds(o, GL)] = plsc.load_gather(sv_v, [idx])
        rk_v[pl.ds(o, GL)] = plsc.load_gather(rid_v, [idx])
        return carry

    lax.fori_loop(0, CHUNK // GL, body, 0)
    pltpu.sync_copy(val_v, sc_out.at[pl.ds(base, CHUNK)])
    pltpu.sync_copy(rk_v, rk_out.at[pl.ds(base, CHUNK)])


def _score_gather(sv_flat, rid_flat, x_flat):
    mesh = plsc.VectorSubcoreMesh(core_axis_name="c", subcore_axis_name="s")
    fn = pl.kernel(
        _gather_body,
        out_type=(
            jax.ShapeDtypeStruct((B * S,), jnp.float32),
            jax.ShapeDtypeStruct((B * S,), jnp.int32),
        ),
        mesh=mesh,
        scratch_types=[
            pltpu.VMEM((P,), jnp.float32),
            pltpu.VMEM((P,), jnp.int32),
            pltpu.VMEM((CHUNK,), jnp.int32),
            pltpu.VMEM((CHUNK,), jnp.float32),
            pltpu.VMEM((CHUNK,), jnp.int32),
        ],
        compiler_params=pltpu.CompilerParams(needs_layout_passes=False),
    )
    return fn(sv_flat, rid_flat, x_flat)


def _topk_body(krank_ref, t_ref, a1m_ref, a1b_ref, a2m_ref, a2b_ref,
               c1m_ref, c1b_ref, c2r_ref, c2b_ref,
               idx_ref, pred_ref, att_ref):
    @pl.when(pl.program_id(0) == 0)
    def _():
        t = t_ref[...]
        a = jnp.dot(t, a1m_ref[...], preferred_element_type=jnp.float32)
        a = jnp.maximum(a + a1b_ref[...], 0.0)
        a = jnp.dot(a, a2m_ref[...], preferred_element_type=jnp.float32)
        att_ref[...] = a + a2b_ref[...]

    krk = krank_ref[...]
    pos = lax.broadcasted_iota(jnp.int32, (R, S), 1)
    key = ((krk >> 10) << 18) | (pos << 10) | (krk & 1023)
    viota = lax.broadcasted_iota(jnp.int32, (R, P), 1)
    m_cnt = jnp.zeros((R, P), jnp.float32)
    for j in range(K):
        kmin = jnp.min(key, axis=1, keepdims=True)
        idx_ref[:, pl.ds(j, 1)] = (kmin >> 10) & 255
        selid = kmin & 1023
        m_cnt = m_cnt + jnp.where(viota == selid, 1.0, 0.0)
        key = jnp.where(key == kmin, IMAX, key)
    pooled = jnp.dot(m_cnt, att_ref[...],
                     preferred_element_type=jnp.float32) * (1.0 / K)
    h = jnp.dot(pooled, c1m_ref[...], preferred_element_type=jnp.float32)
    h = jnp.maximum(h + c1b_ref[...], 0.0)
    logit = jnp.sum(h * c2r_ref[...], axis=1, keepdims=True) + c2b_ref[...]
    pred_ref[...] = jax.nn.sigmoid(logit)


def _topk(krank, table_p, A1, a1r, A2, a2r, C1, c1r, c2row, c2b):
    zero = lambda i: (0, 0)
    return pl.pallas_call(
        _topk_body,
        grid=(B // R,),
        in_specs=[
            pl.BlockSpec((R, S), lambda i: (i, 0)),
            pl.BlockSpec((P, D), zero),
            pl.BlockSpec((D, D), zero),
            pl.BlockSpec((1, D), zero),
            pl.BlockSpec((D, D), zero),
            pl.BlockSpec((1, D), zero),
            pl.BlockSpec((D, D // 2), zero),
            pl.BlockSpec((1, D // 2), zero),
            pl.BlockSpec((1, D // 2), zero),
            pl.BlockSpec((1, 1), zero),
        ],
        out_specs=(
            pl.BlockSpec((R, K), lambda i: (i, 0)),
            pl.BlockSpec((R, 1), lambda i: (i, 0)),
        ),
        out_shape=(
            jax.ShapeDtypeStruct((B, K), jnp.int32),
            jax.ShapeDtypeStruct((B, 1), jnp.float32),
        ),
        scratch_shapes=[pltpu.VMEM((P, D), jnp.float32)],
    )(krank, table_p, A1, a1r, A2, a2r, C1, c1r, c2row, c2b)


def kernel(x, table, W1, b1, W2, b2, W3, b3, A1, a1, A2, a2, C1, c1, C2, c2):
    h = jax.nn.relu(table @ W1 + b1)
    h = jax.nn.relu(h @ W2 + b2)
    sv = jax.nn.sigmoid((h @ W3 + b3).reshape(V))
    sv_flat = jnp.pad(sv, (0, P - V))
    rank = jnp.sum(sv[None, :] > sv[:, None], axis=1).astype(jnp.int32)
    rid = (rank << 10) | jnp.arange(V, dtype=jnp.int32)
    rid_flat = jnp.pad(rid, (0, P - V), constant_values=0x7FFFFFFF)

    scores_flat, krank_flat = _score_gather(sv_flat, rid_flat,
                                            x.reshape(B * S))
    final_scores = scores_flat.reshape(B, S)

    table_p = jnp.pad(table, ((0, P - V), (0, 0)))
    topidx, pred2d = _topk(
        krank_flat.reshape(B, S), table_p, A1, a1.reshape(1, -1),
        A2, a2.reshape(1, -1), C1, c1.reshape(1, -1),
        C2.reshape(1, -1), c2.reshape(1, 1))
    return (pred2d.reshape(B), topidx, final_scores)


# speedup vs baseline: 23.2288x; 1.0437x over previous
import functools

import jax
import jax.numpy as jnp
from jax import lax
from jax.experimental import pallas as pl
from jax.experimental.pallas import tpu as pltpu
from jax.experimental.pallas import tpu_sc as plsc

B, S, V, D = 4096, 200, 1000, 128
K = 20
P = 1024
R = 256
NC, NS = 2, 16
NW = NC * NS
CHUNK = (B * S) // NW
GL = 16
IMAX = 0x7FFFFFFF


def _gather_body(sv_hbm, rid_hbm, x_hbm, sc_out, rk_out,
                 sv_v, rid_v, idx_v, val_v, rk_v):
    wid = lax.axis_index("s") * NC + lax.axis_index("c")
    base = wid * CHUNK
    pltpu.sync_copy(sv_hbm, sv_v)
    pltpu.sync_copy(rid_hbm, rid_v)
    pltpu.sync_copy(x_hbm.at[pl.ds(base, CHUNK)], idx_v)

    def body(i, carry):
        o = i * GL
        idx = idx_v[pl.ds(o, GL)]
        val_v[pl.ds(o, GL)] = plsc.load_gather(sv_v, [idx])
        rk_v[pl.ds(o, GL)] = plsc.load_gather(rid_v, [idx])
        return carry

    lax.fori_loop(0, CHUNK // GL, body, 0)
    pltpu.sync_copy(val_v, sc_out.at[pl.ds(base, CHUNK)])
    pltpu.sync_copy(rk_v, rk_out.at[pl.ds(base, CHUNK)])


def _score_gather(sv_flat, rid_flat, x_flat):
    mesh = plsc.VectorSubcoreMesh(core_axis_name="c", subcore_axis_name="s")
    fn = pl.kernel(
        _gather_body,
        out_type=(
            jax.ShapeDtypeStruct((B * S,), jnp.float32),
            jax.ShapeDtypeStruct((B * S,), jnp.int32),
        ),
        mesh=mesh,
        scratch_types=[
            pltpu.VMEM((P,), jnp.float32),
            pltpu.VMEM((P,), jnp.int32),
            pltpu.VMEM((CHUNK,), jnp.int32),
            pltpu.VMEM((CHUNK,), jnp.float32),
            pltpu.VMEM((CHUNK,), jnp.int32),
        ],
        compiler_params=pltpu.CompilerParams(needs_layout_passes=False),
    )
    return fn(sv_flat, rid_flat, x_flat)


def _topk_body(krank_ref, t_ref, a1m_ref, a1b_ref, a2m_ref, a2b_ref,
               c1m_ref, c1b_ref, c2r_ref, c2b_ref,
               idx_ref, pred_ref, att_ref):
    @pl.when(pl.program_id(0) == 0)
    def _():
        t = t_ref[...]
        a = jnp.dot(t, a1m_ref[...], preferred_element_type=jnp.float32)
        a = jnp.maximum(a + a1b_ref[...], 0.0)
        a = jnp.dot(a, a2m_ref[...], preferred_element_type=jnp.float32)
        att_ref[...] = a + a2b_ref[...]

    krk = krank_ref[...]
    pos = lax.broadcasted_iota(jnp.int32, (R, S), 1)
    key = ((krk >> 10) << 18) | (pos << 10) | (krk & 1023)
    viota = lax.broadcasted_iota(jnp.int32, (R, P), 1).astype(jnp.int16)
    one16 = jnp.zeros((R, P), jnp.int16) + 1
    m_cnt = jnp.zeros((R, P), jnp.int16)
    for j in range(K):
        kmin = jnp.min(key, axis=1, keepdims=True)
        idx_ref[:, pl.ds(j, 1)] = (kmin >> 10) & 255
        selid = (kmin & 1023).astype(jnp.int16)
        m_cnt = m_cnt + jnp.where(viota == selid, one16, 0)
        key = jnp.where(key == kmin, IMAX, key)
    pooled = jnp.dot(m_cnt.astype(jnp.float32), att_ref[...],
                     preferred_element_type=jnp.float32) * (1.0 / K)
    h = jnp.dot(pooled, c1m_ref[...], preferred_element_type=jnp.float32)
    h = jnp.maximum(h + c1b_ref[...], 0.0)
    logit = jnp.sum(h * c2r_ref[...], axis=1, keepdims=True) + c2b_ref[...]
    pred_ref[...] = jax.nn.sigmoid(logit)


def _topk(krank, table_p, A1, a1r, A2, a2r, C1, c1r, c2row, c2b):
    zero = lambda i: (0, 0)
    return pl.pallas_call(
        _topk_body,
        grid=(B // R,),
        in_specs=[
            pl.BlockSpec((R, S), lambda i: (i, 0)),
            pl.BlockSpec((P, D), zero),
            pl.BlockSpec((D, D), zero),
            pl.BlockSpec((1, D), zero),
            pl.BlockSpec((D, D), zero),
            pl.BlockSpec((1, D), zero),
            pl.BlockSpec((D, D // 2), zero),
            pl.BlockSpec((1, D // 2), zero),
            pl.BlockSpec((1, D // 2), zero),
            pl.BlockSpec((1, 1), zero),
        ],
        out_specs=(
            pl.BlockSpec((R, K), lambda i: (i, 0)),
            pl.BlockSpec((R, 1), lambda i: (i, 0)),
        ),
        out_shape=(
            jax.ShapeDtypeStruct((B, K), jnp.int32),
            jax.ShapeDtypeStruct((B, 1), jnp.float32),
        ),
        scratch_shapes=[pltpu.VMEM((P, D), jnp.float32)],
    )(krank, table_p, A1, a1r, A2, a2r, C1, c1r, c2row, c2b)


def kernel(x, table, W1, b1, W2, b2, W3, b3, A1, a1, A2, a2, C1, c1, C2, c2):
    h = jax.nn.relu(table @ W1 + b1)
    h = jax.nn.relu(h @ W2 + b2)
    sv = jax.nn.sigmoid((h @ W3 + b3).reshape(V))
    sv_flat = jnp.pad(sv, (0, P - V))
    rank = jnp.sum(sv[None, :] > sv[:, None], axis=1).astype(jnp.int32)
    rid = (rank << 10) | jnp.arange(V, dtype=jnp.int32)
    rid_flat = jnp.pad(rid, (0, P - V), constant_values=0x7FFFFFFF)

    scores_flat, krank_flat = _score_gather(sv_flat, rid_flat,
                                            x.reshape(B * S))
    final_scores = scores_flat.reshape(B, S)

    table_p = jnp.pad(table, ((0, P - V), (0, 0)))
    topidx, pred2d = _topk(
        krank_flat.reshape(B, S), table_p, A1, a1.reshape(1, -1),
        A2, a2.reshape(1, -1), C1, c1.reshape(1, -1),
        C2.reshape(1, -1), c2.reshape(1, 1))
    return (pred2d.reshape(B), topidx, final_scores)


# speedup vs baseline: 25.2336x; 1.0863x over previous
import functools

import jax
import jax.numpy as jnp
from jax import lax
from jax.experimental import pallas as pl
from jax.experimental.pallas import tpu as pltpu
from jax.experimental.pallas import tpu_sc as plsc

B, S, V, D = 4096, 200, 1000, 128
K = 20
P = 1024
R = 256
NC, NS = 2, 16
NW = NC * NS
CHUNK = (B * S) // NW
GL = 16
IMAX = 0x7FFFFFFF


def _gather_body(sv_hbm, rid_hbm, x_hbm, sc_out, rk_out,
                 sv_v, rid_v, idx_v, val_v, rk_v):
    wid = lax.axis_index("s") * NC + lax.axis_index("c")
    base = wid * CHUNK
    pltpu.sync_copy(sv_hbm, sv_v)
    pltpu.sync_copy(rid_hbm, rid_v)
    pltpu.sync_copy(x_hbm.at[pl.ds(base, CHUNK)], idx_v)

    @plsc.parallel_loop(0, CHUNK // GL, unroll=8)
    def body(i):
        o = i * GL
        idx = idx_v[pl.ds(o, GL)]
        val_v[pl.ds(o, GL)] = plsc.load_gather(sv_v, [idx])
        rk_v[pl.ds(o, GL)] = plsc.load_gather(rid_v, [idx])
    pltpu.sync_copy(val_v, sc_out.at[pl.ds(base, CHUNK)])
    pltpu.sync_copy(rk_v, rk_out.at[pl.ds(base, CHUNK)])


def _score_gather(sv_flat, rid_flat, x_flat):
    mesh = plsc.VectorSubcoreMesh(core_axis_name="c", subcore_axis_name="s")
    fn = pl.kernel(
        _gather_body,
        out_type=(
            jax.ShapeDtypeStruct((B * S,), jnp.float32),
            jax.ShapeDtypeStruct((B * S,), jnp.int32),
        ),
        mesh=mesh,
        scratch_types=[
            pltpu.VMEM((P,), jnp.float32),
            pltpu.VMEM((P,), jnp.int32),
            pltpu.VMEM((CHUNK,), jnp.int32),
            pltpu.VMEM((CHUNK,), jnp.float32),
            pltpu.VMEM((CHUNK,), jnp.int32),
        ],
        compiler_params=pltpu.CompilerParams(needs_layout_passes=False),
    )
    return fn(sv_flat, rid_flat, x_flat)


def _topk_body(krank_ref, t_ref, a1m_ref, a1b_ref, a2m_ref, a2b_ref,
               c1m_ref, c1b_ref, c2r_ref, c2b_ref,
               idx_ref, pred_ref, att_ref):
    @pl.when(pl.program_id(0) == 0)
    def _():
        t = t_ref[...]
        a = jnp.dot(t, a1m_ref[...], preferred_element_type=jnp.float32)
        a = jnp.maximum(a + a1b_ref[...], 0.0)
        a = jnp.dot(a, a2m_ref[...], preferred_element_type=jnp.float32)
        att_ref[...] = a + a2b_ref[...]

    krk = krank_ref[...]
    pos = lax.broadcasted_iota(jnp.int32, (R, S), 1)
    key = ((krk >> 10) << 18) | (pos << 10) | (krk & 1023)
    viota = lax.broadcasted_iota(jnp.int32, (R, P), 1).astype(jnp.int16)
    one16 = jnp.zeros((R, P), jnp.int16) + 1
    m_cnt = jnp.zeros((R, P), jnp.int16)
    for j in range(K):
        kmin = jnp.min(key, axis=1, keepdims=True)
        idx_ref[:, pl.ds(j, 1)] = (kmin >> 10) & 255
        selid = (kmin & 1023).astype(jnp.int16)
        m_cnt = m_cnt + jnp.where(viota == selid, one16, 0)
        key = jnp.where(key == kmin, IMAX, key)
    pooled = jnp.dot(m_cnt.astype(jnp.float32), att_ref[...],
                     preferred_element_type=jnp.float32) * (1.0 / K)
    h = jnp.dot(pooled, c1m_ref[...], preferred_element_type=jnp.float32)
    h = jnp.maximum(h + c1b_ref[...], 0.0)
    logit = jnp.sum(h * c2r_ref[...], axis=1, keepdims=True) + c2b_ref[...]
    pred_ref[...] = jax.nn.sigmoid(logit)


def _topk(krank, table_p, A1, a1r, A2, a2r, C1, c1r, c2row, c2b):
    zero = lambda i: (0, 0)
    return pl.pallas_call(
        _topk_body,
        grid=(B // R,),
        in_specs=[
            pl.BlockSpec((R, S), lambda i: (i, 0)),
            pl.BlockSpec((P, D), zero),
            pl.BlockSpec((D, D), zero),
            pl.BlockSpec((1, D), zero),
            pl.BlockSpec((D, D), zero),
            pl.BlockSpec((1, D), zero),
            pl.BlockSpec((D, D // 2), zero),
            pl.BlockSpec((1, D // 2), zero),
            pl.BlockSpec((1, D // 2), zero),
            pl.BlockSpec((1, 1), zero),
        ],
        out_specs=(
            pl.BlockSpec((R, K), lambda i: (i, 0)),
            pl.BlockSpec((R, 1), lambda i: (i, 0)),
        ),
        out_shape=(
            jax.ShapeDtypeStruct((B, K), jnp.int32),
            jax.ShapeDtypeStruct((B, 1), jnp.float32),
        ),
        scratch_shapes=[pltpu.VMEM((P, D), jnp.float32)],
    )(krank, table_p, A1, a1r, A2, a2r, C1, c1r, c2row, c2b)


def kernel(x, table, W1, b1, W2, b2, W3, b3, A1, a1, A2, a2, C1, c1, C2, c2):
    h = jax.nn.relu(table @ W1 + b1)
    h = jax.nn.relu(h @ W2 + b2)
    sv = jax.nn.sigmoid((h @ W3 + b3).reshape(V))
    sv_flat = jnp.pad(sv, (0, P - V))
    rank = jnp.sum(sv[None, :] > sv[:, None], axis=1).astype(jnp.int32)
    rid = (rank << 10) | jnp.arange(V, dtype=jnp.int32)
    rid_flat = jnp.pad(rid, (0, P - V), constant_values=0x7FFFFFFF)

    scores_flat, krank_flat = _score_gather(sv_flat, rid_flat,
                                            x.reshape(B * S))
    final_scores = scores_flat.reshape(B, S)

    table_p = jnp.pad(table, ((0, P - V), (0, 0)))
    topidx, pred2d = _topk(
        krank_flat.reshape(B, S), table_p, A1, a1.reshape(1, -1),
        A2, a2.reshape(1, -1), C1, c1.reshape(1, -1),
        C2.reshape(1, -1), c2.reshape(1, 1))
    return (pred2d.reshape(B), topidx, final_scores)


# speedup vs baseline: 28.2475x; 1.1194x over previous
import functools

import jax
import jax.numpy as jnp
from jax import lax
from jax.experimental import pallas as pl
from jax.experimental.pallas import tpu as pltpu
from jax.experimental.pallas import tpu_sc as plsc

B, S, V, D = 4096, 200, 1000, 128
K = 20
P = 1024
R = 1024
NC, NS = 2, 16
NW = NC * NS
CHUNK = (B * S) // NW
GL = 16
IMAX = 0x7FFFFFFF


def _gather_body(sv_hbm, rid_hbm, x_hbm, sc_out, rk_out,
                 sv_v, rid_v, idx_v, val_v, rk_v):
    wid = lax.axis_index("s") * NC + lax.axis_index("c")
    base = wid * CHUNK
    pltpu.sync_copy(sv_hbm, sv_v)
    pltpu.sync_copy(rid_hbm, rid_v)
    pltpu.sync_copy(x_hbm.at[pl.ds(base, CHUNK)], idx_v)

    @plsc.parallel_loop(0, CHUNK // GL, unroll=8)
    def body(i):
        o = i * GL
        idx = idx_v[pl.ds(o, GL)]
        val_v[pl.ds(o, GL)] = plsc.load_gather(sv_v, [idx])
        rk_v[pl.ds(o, GL)] = plsc.load_gather(rid_v, [idx])
    pltpu.sync_copy(val_v, sc_out.at[pl.ds(base, CHUNK)])
    pltpu.sync_copy(rk_v, rk_out.at[pl.ds(base, CHUNK)])


def _score_gather(sv_flat, rid_flat, x_flat):
    mesh = plsc.VectorSubcoreMesh(core_axis_name="c", subcore_axis_name="s")
    fn = pl.kernel(
        _gather_body,
        out_type=(
            jax.ShapeDtypeStruct((B * S,), jnp.float32),
            jax.ShapeDtypeStruct((B * S,), jnp.int32),
        ),
        mesh=mesh,
        scratch_types=[
            pltpu.VMEM((P,), jnp.float32),
            pltpu.VMEM((P,), jnp.int32),
            pltpu.VMEM((CHUNK,), jnp.int32),
            pltpu.VMEM((CHUNK,), jnp.float32),
            pltpu.VMEM((CHUNK,), jnp.int32),
        ],
        compiler_params=pltpu.CompilerParams(needs_layout_passes=False),
    )
    return fn(sv_flat, rid_flat, x_flat)


def _topk_body(krank_ref, t_ref, a1m_ref, a1b_ref, a2m_ref, a2b_ref,
               c1m_ref, c1b_ref, c2r_ref, c2b_ref,
               idx_ref, pred_ref, att_ref):
    @pl.when(pl.program_id(0) == 0)
    def _():
        t = t_ref[...]
        a = jnp.dot(t, a1m_ref[...], preferred_element_type=jnp.float32)
        a = jnp.maximum(a + a1b_ref[...], 0.0)
        a = jnp.dot(a, a2m_ref[...], preferred_element_type=jnp.float32)
        att_ref[...] = a + a2b_ref[...]

    krk = krank_ref[...]
    pos = lax.broadcasted_iota(jnp.int32, (R, S), 1)
    key = ((krk >> 10) << 18) | (pos << 10) | (krk & 1023)
    viota = lax.broadcasted_iota(jnp.int32, (R, P), 1).astype(jnp.int16)
    m_cnt = jnp.zeros((R, P), jnp.int16)
    for j in range(K):
        kmin = jnp.min(key, axis=1, keepdims=True)
        idx_ref[:, pl.ds(j, 1)] = (kmin >> 10) & 255
        selid = (kmin & 1023).astype(jnp.int16)
        m_cnt = m_cnt + jnp.where(viota == selid,
                                  jnp.int16(1), jnp.int16(0))
        key = jnp.where(key == kmin, IMAX, key)
    pooled = jnp.dot(m_cnt.astype(jnp.float32), att_ref[...],
                     preferred_element_type=jnp.float32) * (1.0 / K)
    h = jnp.dot(pooled, c1m_ref[...], preferred_element_type=jnp.float32)
    h = jnp.maximum(h + c1b_ref[...], 0.0)
    logit = jnp.sum(h * c2r_ref[...], axis=1, keepdims=True) + c2b_ref[...]
    pred_ref[...] = jax.nn.sigmoid(logit)


def _topk(krank, table_p, A1, a1r, A2, a2r, C1, c1r, c2row, c2b):
    zero = lambda i: (0, 0)
    return pl.pallas_call(
        _topk_body,
        grid=(B // R,),
        in_specs=[
            pl.BlockSpec((R, S), lambda i: (i, 0)),
            pl.BlockSpec((P, D), zero),
            pl.BlockSpec((D, D), zero),
            pl.BlockSpec((1, D), zero),
            pl.BlockSpec((D, D), zero),
            pl.BlockSpec((1, D), zero),
            pl.BlockSpec((D, D // 2), zero),
            pl.BlockSpec((1, D // 2), zero),
            pl.BlockSpec((1, D // 2), zero),
            pl.BlockSpec((1, 1), zero),
        ],
        out_specs=(
            pl.BlockSpec((R, K), lambda i: (i, 0)),
            pl.BlockSpec((R, 1), lambda i: (i, 0)),
        ),
        out_shape=(
            jax.ShapeDtypeStruct((B, K), jnp.int32),
            jax.ShapeDtypeStruct((B, 1), jnp.float32),
        ),
        scratch_shapes=[pltpu.VMEM((P, D), jnp.float32)],
    )(krank, table_p, A1, a1r, A2, a2r, C1, c1r, c2row, c2b)


def kernel(x, table, W1, b1, W2, b2, W3, b3, A1, a1, A2, a2, C1, c1, C2, c2):
    h = jax.nn.relu(table @ W1 + b1)
    h = jax.nn.relu(h @ W2 + b2)
    sv = jax.nn.sigmoid((h @ W3 + b3).reshape(V))
    sv_flat = jnp.pad(sv, (0, P - V))
    rank = jnp.sum(sv[None, :] > sv[:, None], axis=1).astype(jnp.int32)
    rid = (rank << 10) | jnp.arange(V, dtype=jnp.int32)
    rid_flat = jnp.pad(rid, (0, P - V), constant_values=0x7FFFFFFF)

    scores_flat, krank_flat = _score_gather(sv_flat, rid_flat,
                                            x.reshape(B * S))
    final_scores = scores_flat.reshape(B, S)

    table_p = jnp.pad(table, ((0, P - V), (0, 0)))
    topidx, pred2d = _topk(
        krank_flat.reshape(B, S), table_p, A1, a1.reshape(1, -1),
        A2, a2.reshape(1, -1), C1, c1.reshape(1, -1),
        C2.reshape(1, -1), c2.reshape(1, 1))
    return (pred2d.reshape(B), topidx, final_scores)


# speedup vs baseline: 28.2517x; 1.0001x over previous
import functools

import jax
import jax.numpy as jnp
from jax import lax
from jax.experimental import pallas as pl
from jax.experimental.pallas import tpu as pltpu
from jax.experimental.pallas import tpu_sc as plsc

B, S, V, D = 4096, 200, 1000, 128
K = 20
P = 1024
R = 1024
NC, NS = 2, 16
NW = NC * NS
CHUNK = (B * S) // NW
GL = 16
IMAX = 0x7FFFFFFF


def _gather_body(sv_hbm, rid_hbm, x_hbm, sc_out, rk_out,
                 sv_v, rid_v, idx_v, val_v, rk_v):
    wid = lax.axis_index("s") * NC + lax.axis_index("c")
    base = wid * CHUNK
    pltpu.sync_copy(sv_hbm, sv_v)
    pltpu.sync_copy(rid_hbm, rid_v)
    pltpu.sync_copy(x_hbm.at[pl.ds(base, CHUNK)], idx_v)

    @plsc.parallel_loop(0, CHUNK // GL, unroll=8)
    def body(i):
        o = i * GL
        idx = idx_v[pl.ds(o, GL)]
        val_v[pl.ds(o, GL)] = plsc.load_gather(sv_v, [idx])
        rk_v[pl.ds(o, GL)] = plsc.load_gather(rid_v, [idx])
    pltpu.sync_copy(val_v, sc_out.at[pl.ds(base, CHUNK)])
    pltpu.sync_copy(rk_v, rk_out.at[pl.ds(base, CHUNK)])


def _score_gather(sv_flat, rid_flat, x_flat):
    mesh = plsc.VectorSubcoreMesh(core_axis_name="c", subcore_axis_name="s")
    fn = pl.kernel(
        _gather_body,
        out_type=(
            jax.ShapeDtypeStruct((B * S,), jnp.float32),
            jax.ShapeDtypeStruct((B * S,), jnp.int32),
        ),
        mesh=mesh,
        scratch_types=[
            pltpu.VMEM((P,), jnp.float32),
            pltpu.VMEM((P,), jnp.int32),
            pltpu.VMEM((CHUNK,), jnp.int32),
            pltpu.VMEM((CHUNK,), jnp.float32),
            pltpu.VMEM((CHUNK,), jnp.int32),
        ],
        compiler_params=pltpu.CompilerParams(
            needs_layout_passes=False,
            disable_bounds_checks=True,
            disable_semaphore_checks=True),
    )
    return fn(sv_flat, rid_flat, x_flat)


def _topk_body(krank_ref, t_ref, a1m_ref, a1b_ref, a2m_ref, a2b_ref,
               c1m_ref, c1b_ref, c2r_ref, c2b_ref,
               idx_ref, pred_ref, att_ref):
    @pl.when(pl.program_id(0) == 0)
    def _():
        t = t_ref[...]
        a = jnp.dot(t, a1m_ref[...], preferred_element_type=jnp.float32)
        a = jnp.maximum(a + a1b_ref[...], 0.0)
        a = jnp.dot(a, a2m_ref[...], preferred_element_type=jnp.float32)
        att_ref[...] = a + a2b_ref[...]

    krk = krank_ref[...]
    pos = lax.broadcasted_iota(jnp.int32, (R, S), 1)
    key = ((krk >> 10) << 18) | (pos << 10) | (krk & 1023)
    viota = lax.broadcasted_iota(jnp.int32, (R, P), 1).astype(jnp.int16)
    m_cnt = jnp.zeros((R, P), jnp.int16)
    for j in range(K):
        kmin = jnp.min(key, axis=1, keepdims=True)
        idx_ref[:, pl.ds(j, 1)] = (kmin >> 10) & 255
        selid = (kmin & 1023).astype(jnp.int16)
        m_cnt = m_cnt + jnp.where(viota == selid,
                                  jnp.int16(1), jnp.int16(0))
        key = jnp.where(key == kmin, IMAX, key)
    pooled = jnp.dot(m_cnt.astype(jnp.float32), att_ref[...],
                     preferred_element_type=jnp.float32) * (1.0 / K)
    h = jnp.dot(pooled, c1m_ref[...], preferred_element_type=jnp.float32)
    h = jnp.maximum(h + c1b_ref[...], 0.0)
    logit = jnp.sum(h * c2r_ref[...], axis=1, keepdims=True) + c2b_ref[...]
    pred_ref[...] = jax.nn.sigmoid(logit)


def _topk(krank, table_p, A1, a1r, A2, a2r, C1, c1r, c2row, c2b):
    zero = lambda i: (0, 0)
    return pl.pallas_call(
        _topk_body,
        grid=(B // R,),
        in_specs=[
            pl.BlockSpec((R, S), lambda i: (i, 0)),
            pl.BlockSpec((P, D), zero),
            pl.BlockSpec((D, D), zero),
            pl.BlockSpec((1, D), zero),
            pl.BlockSpec((D, D), zero),
            pl.BlockSpec((1, D), zero),
            pl.BlockSpec((D, D // 2), zero),
            pl.BlockSpec((1, D // 2), zero),
            pl.BlockSpec((1, D // 2), zero),
            pl.BlockSpec((1, 1), zero),
        ],
        out_specs=(
            pl.BlockSpec((R, K), lambda i: (i, 0)),
            pl.BlockSpec((R, 1), lambda i: (i, 0)),
        ),
        out_shape=(
            jax.ShapeDtypeStruct((B, K), jnp.int32),
            jax.ShapeDtypeStruct((B, 1), jnp.float32),
        ),
        scratch_shapes=[pltpu.VMEM((P, D), jnp.float32)],
        compiler_params=pltpu.CompilerParams(disable_bounds_checks=True),
    )(krank, table_p, A1, a1r, A2, a2r, C1, c1r, c2row, c2b)


def kernel(x, table, W1, b1, W2, b2, W3, b3, A1, a1, A2, a2, C1, c1, C2, c2):
    h = jax.nn.relu(table @ W1 + b1)
    h = jax.nn.relu(h @ W2 + b2)
    sv = jax.nn.sigmoid((h @ W3 + b3).reshape(V))
    sv_flat = jnp.pad(sv, (0, P - V))
    rank = jnp.sum(sv[None, :] > sv[:, None], axis=1).astype(jnp.int32)
    rid = (rank << 10) | jnp.arange(V, dtype=jnp.int32)
    rid_flat = jnp.pad(rid, (0, P - V), constant_values=0x7FFFFFFF)

    scores_flat, krank_flat = _score_gather(sv_flat, rid_flat,
                                            x.reshape(B * S))
    final_scores = scores_flat.reshape(B, S)

    table_p = jnp.pad(table, ((0, P - V), (0, 0)))
    topidx, pred2d = _topk(
        krank_flat.reshape(B, S), table_p, A1, a1.reshape(1, -1),
        A2, a2.reshape(1, -1), C1, c1.reshape(1, -1),
        C2.reshape(1, -1), c2.reshape(1, 1))
    return (pred2d.reshape(B), topidx, final_scores)


# speedup vs baseline: 28.3106x; 1.0021x over previous
import jax
import jax.numpy as jnp
from jax import lax
from jax.experimental import pallas as pl
from jax.experimental.pallas import tpu as pltpu
from jax.experimental.pallas import tpu_sc as plsc

B, S, V, D = 4096, 200, 1000, 128
K = 20
P = 1024
R = 1024
NC, NS = 2, 16
NW = NC * NS
CHUNK = (B * S) // NW
GL = 16
IMAX = 0x7FFFFFFF


def _gather_body(sv_hbm, rid_hbm, x_hbm, sc_out, rk_out,
                 sv_v, rid_v, idx_v, val_v, rk_v):
    wid = lax.axis_index("s") * NC + lax.axis_index("c")
    base = wid * CHUNK
    pltpu.sync_copy(sv_hbm, sv_v)
    pltpu.sync_copy(rid_hbm, rid_v)
    pltpu.sync_copy(x_hbm.at[pl.ds(base, CHUNK)], idx_v)

    @plsc.parallel_loop(0, CHUNK // GL, unroll=8)
    def body(i):
        o = i * GL
        idx = idx_v[pl.ds(o, GL)]
        val_v[pl.ds(o, GL)] = plsc.load_gather(sv_v, [idx])
        rk_v[pl.ds(o, GL)] = plsc.load_gather(rid_v, [idx])
    pltpu.sync_copy(val_v, sc_out.at[pl.ds(base, CHUNK)])
    pltpu.sync_copy(rk_v, rk_out.at[pl.ds(base, CHUNK)])


def _score_gather(sv_flat, rid_flat, x_flat):
    mesh = plsc.VectorSubcoreMesh(core_axis_name="c", subcore_axis_name="s")
    fn = pl.kernel(
        _gather_body,
        out_type=(
            jax.ShapeDtypeStruct((B * S,), jnp.float32),
            jax.ShapeDtypeStruct((B * S,), jnp.int32),
        ),
        mesh=mesh,
        scratch_types=[
            pltpu.VMEM((P,), jnp.float32),
            pltpu.VMEM((P,), jnp.int32),
            pltpu.VMEM((CHUNK,), jnp.int32),
            pltpu.VMEM((CHUNK,), jnp.float32),
            pltpu.VMEM((CHUNK,), jnp.int32),
        ],
        compiler_params=pltpu.CompilerParams(
            needs_layout_passes=False,
            disable_bounds_checks=True,
            disable_semaphore_checks=True),
    )
    return fn(sv_flat, rid_flat, x_flat)


def _topk_body(krank_ref, t_ref, a1m_ref, a1b_ref, a2m_ref, a2b_ref,
               c1m_ref, c1b_ref, c2r_ref, c2b_ref,
               idx_ref, pred_ref, att_ref):
    @pl.when(pl.program_id(0) == 0)
    def _():
        t = t_ref[...]
        a = jnp.dot(t, a1m_ref[...], preferred_element_type=jnp.float32)
        a = jnp.maximum(a + a1b_ref[...], 0.0)
        a = jnp.dot(a, a2m_ref[...], preferred_element_type=jnp.float32)
        att_ref[...] = a + a2b_ref[...]

    krk = krank_ref[...]
    pos = lax.broadcasted_iota(jnp.int32, (R, S), 1)
    key = ((krk >> 10) << 18) | (pos << 10) | (krk & 1023)
    viota = lax.broadcasted_iota(jnp.int32, (R, P), 1).astype(jnp.int16)
    m_cnt = jnp.zeros((R, P), jnp.int16)
    for j in range(K):
        kmin = jnp.min(key, axis=1, keepdims=True)
        idx_ref[:, pl.ds(j, 1)] = (kmin >> 10) & 255
        selid = (kmin & 1023).astype(jnp.int16)
        m_cnt = m_cnt + jnp.where(viota == selid,
                                  jnp.int16(1), jnp.int16(0))
        key = jnp.where(key == kmin, IMAX, key)
    pooled = jnp.dot(m_cnt.astype(jnp.float32), att_ref[...],
                     preferred_element_type=jnp.float32) * (1.0 / K)
    h = jnp.dot(pooled, c1m_ref[...], preferred_element_type=jnp.float32)
    h = jnp.maximum(h + c1b_ref[...], 0.0)
    logit = jnp.sum(h * c2r_ref[...], axis=1, keepdims=True) + c2b_ref[...]
    pred_ref[...] = jax.nn.sigmoid(logit)


def _topk(krank, table_p, A1, a1r, A2, a2r, C1, c1r, c2row, c2b):
    zero = lambda i: (0, 0)
    return pl.pallas_call(
        _topk_body,
        grid=(B // R,),
        in_specs=[
            pl.BlockSpec((R, S), lambda i: (i, 0)),
            pl.BlockSpec((P, D), zero),
            pl.BlockSpec((D, D), zero),
            pl.BlockSpec((1, D), zero),
            pl.BlockSpec((D, D), zero),
            pl.BlockSpec((1, D), zero),
            pl.BlockSpec((D, D // 2), zero),
            pl.BlockSpec((1, D // 2), zero),
            pl.BlockSpec((1, D // 2), zero),
            pl.BlockSpec((1, 1), zero),
        ],
        out_specs=(
            pl.BlockSpec((R, K), lambda i: (i, 0)),
            pl.BlockSpec((R, 1), lambda i: (i, 0)),
        ),
        out_shape=(
            jax.ShapeDtypeStruct((B, K), jnp.int32),
            jax.ShapeDtypeStruct((B, 1), jnp.float32),
        ),
        scratch_shapes=[pltpu.VMEM((P, D), jnp.float32)],
        compiler_params=pltpu.CompilerParams(disable_bounds_checks=True),
    )(krank, table_p, A1, a1r, A2, a2r, C1, c1r, c2row, c2b)


def kernel(x, table, W1, b1, W2, b2, W3, b3, A1, a1, A2, a2, C1, c1, C2, c2):
    h = jax.nn.relu(table @ W1 + b1)
    h = jax.nn.relu(h @ W2 + b2)
    sv = jax.nn.sigmoid((h @ W3 + b3).reshape(V))
    sv_flat = jnp.pad(sv, (0, P - V))
    rank = jnp.sum(sv[None, :] > sv[:, None], axis=1).astype(jnp.int32)
    rid = (rank << 10) | jnp.arange(V, dtype=jnp.int32)
    rid_flat = jnp.pad(rid, (0, P - V), constant_values=0x7FFFFFFF)

    scores_flat, krank_flat = _score_gather(sv_flat, rid_flat,
                                            x.reshape(B * S))
    final_scores = scores_flat.reshape(B, S)

    table_p = jnp.pad(table, ((0, P - V), (0, 0)))
    topidx, pred2d = _topk(
        krank_flat.reshape(B, S), table_p, A1, a1.reshape(1, -1),
        A2, a2.reshape(1, -1), C1, c1.reshape(1, -1),
        C2.reshape(1, -1), c2.reshape(1, 1))
    return (pred2d.reshape(B), topidx, final_scores)
